# Initial kernel scaffold; baseline (speedup 1.0000x reference)
import jax
import jax.numpy as jnp
from jax.experimental import pallas as pl


def kernel(h, Wr, br, W1, b1, W2, b2):
    raise NotImplementedError("write your pallas kernel here")


# speedup vs baseline: 1.3274x; 1.3274x over previous
import functools

import jax
import jax.numpy as jnp
from jax.experimental import pallas as pl
from jax.experimental.pallas import tpu as pltpu

TAU = 0.05


def _moe_block_kernel(h_ref, wr_ref, br_ref, w1_ref, b1_ref, w2_ref, b2_ref,
                      out_ref, *, n_experts):
    h = h_ref[...]
    h_bf = h.astype(jnp.bfloat16)
    logits = jax.lax.dot_general(
        h_bf, wr_ref[...].astype(jnp.bfloat16), (((1,), (0,)), ((), ())),
        preferred_element_type=jnp.float32) + br_ref[...]
    logits = logits - jnp.max(logits, axis=1, keepdims=True)
    expw = jnp.exp(logits)
    weights = expw / jnp.sum(expw, axis=1, keepdims=True)
    weights = jnp.where(weights > TAU, weights, 0.0)

    acc = jnp.zeros(out_ref.shape, jnp.float32)
    for e in range(n_experts):
        hidden = jax.lax.dot_general(
            h_bf, w1_ref[e], (((1,), (0,)), ((), ())),
            preferred_element_type=jnp.float32)
        hidden = hidden + b1_ref[e:e + 1, :]
        act = jax.nn.gelu(hidden).astype(jnp.bfloat16)
        out_e = jax.lax.dot_general(
            act, w2_ref[e], (((1,), (0,)), ((), ())),
            preferred_element_type=jnp.float32)
        out_e = out_e + b2_ref[e:e + 1, :]
        acc = acc + weights[:, e:e + 1] * out_e
    out_ref[...] = acc


@jax.jit
def kernel(h, Wr, br, W1, b1, W2, b2):
    T, D = h.shape
    E = Wr.shape[1]
    F = W1.shape[2]
    BT = 256
    w1_bf = W1.astype(jnp.bfloat16)
    w2_bf = W2.astype(jnp.bfloat16)
    br2 = br.reshape(1, E)
    grid = (T // BT,)
    return pl.pallas_call(
        functools.partial(_moe_block_kernel, n_experts=E),
        grid=grid,
        in_specs=[
            pl.BlockSpec((BT, D), lambda i: (i, 0)),
            pl.BlockSpec((D, E), lambda i: (0, 0)),
            pl.BlockSpec((1, E), lambda i: (0, 0)),
            pl.BlockSpec((E, D, F), lambda i: (0, 0, 0)),
            pl.BlockSpec((E, F), lambda i: (0, 0)),
            pl.BlockSpec((E, F, D), lambda i: (0, 0, 0)),
            pl.BlockSpec((E, D), lambda i: (0, 0)),
        ],
        out_specs=pl.BlockSpec((BT, D), lambda i: (i, 0)),
        out_shape=jax.ShapeDtypeStruct((T, D), jnp.float32),
        compiler_params=pltpu.CompilerParams(
            dimension_semantics=("arbitrary",),
        ),
    )(h, Wr, br2, w1_bf, b1, w2_bf, b2)
